# Initial kernel scaffold; baseline (speedup 1.0000x reference)
import jax
import jax.numpy as jnp
from jax.experimental import pallas as pl


def kernel(z, codebook):
    raise NotImplementedError("write your pallas kernel here")


# speedup vs baseline: 1.0398x; 1.0398x over previous
import jax
import jax.numpy as jnp
from jax.experimental import pallas as pl

_CB = 8192
_D = 32
_TB = 1024
_CH = 2048
_NCH = _CB // _CH
_BETA = 0.25


def _argmin_body(cn_ref, zn_ref, cb_ref, zb_ref, idx_ref):
    zbv = zb_ref[...]

    def chunk(j, carry):
        bv, bi = carry
        cbc = cb_ref[pl.ds(j * _CH, _CH), :]
        mm = jax.lax.dot_general(
            cbc, zbv, (((1,), (1,)), ((), ())),
            preferred_element_type=jnp.float32)
        t = cn_ref[pl.ds(j * _CH, _CH), :] + zn_ref[0]
        d = t - 2.0 * mm
        m = jnp.min(d, axis=0, keepdims=True)
        rid = jax.lax.broadcasted_iota(jnp.int32, (_CH, _TB), 0) + j * _CH
        cand = jnp.where(d == m, rid, _CB)
        ci = jnp.min(cand, axis=0, keepdims=True)
        keep = bv <= m
        nv = jnp.where(keep, bv, m)
        ni = jnp.where(keep, bi, ci)
        nv = nv.astype(jnp.bfloat16).astype(jnp.float32)
        return nv, ni

    bv0 = jnp.full((1, _TB), jnp.inf, jnp.float32)
    bi0 = jnp.zeros((1, _TB), jnp.int32)
    _, bi = jax.lax.fori_loop(0, _NCH, chunk, (bv0, bi0))
    idx_ref[...] = bi.reshape(1, 1, _TB)


def _argmin_call(cn2, zn3, cbb, zb):
    return pl.pallas_call(
        _argmin_body,
        grid=(8,),
        in_specs=[
            pl.BlockSpec((_CB, 1), lambda i: (0, 0)),
            pl.BlockSpec((1, 1, _TB), lambda i: (i, 0, 0)),
            pl.BlockSpec((_CB, _D), lambda i: (0, 0)),
            pl.BlockSpec((_TB, _D), lambda i: (i, 0)),
        ],
        out_specs=pl.BlockSpec((1, 1, _TB), lambda i: (i, 0, 0)),
        out_shape=jax.ShapeDtypeStruct((8, 1, _TB), jnp.int32),
    )(cn2, zn3, cbb, zb)


def kernel(z, codebook):
    z_flat = z.reshape(-1, _D)
    zb = z_flat.astype(jnp.bfloat16)
    cbb = codebook.astype(jnp.bfloat16)
    zn = jnp.sum(z_flat ** 2, axis=1)
    cn = jnp.sum(codebook ** 2, axis=1)

    idx = _argmin_call(cn.reshape(_CB, 1), zn.reshape(8, 1, _TB),
                       cbb, zb).reshape(-1)

    z_q = jnp.take(codebook, idx, axis=0).reshape(z.shape)
    counts = jnp.bincount(idx, minlength=_CB, length=_CB)
    e_mean = counts.astype(jnp.float32) / idx.size
    perplexity = jnp.exp(-jnp.sum(e_mean * jnp.log(e_mean + 1e-8)))
    commit_loss = (_BETA * jnp.mean((jax.lax.stop_gradient(z_q) - z) ** 2)
                   + jnp.mean((z_q - jax.lax.stop_gradient(z)) ** 2))
    z_q_st = z + jax.lax.stop_gradient(z_q - z)
    return (z_q_st, commit_loss, perplexity)
